# baseline (speedup 1.0000x reference)
import functools

import jax
import jax.numpy as jnp
from jax import lax
from jax.experimental import pallas as pl
from jax.experimental.pallas import tpu as pltpu
from jax.experimental.pallas import tpu_sc as plsc

NC = 2
NS = 16
LANES = 16
NW = NC * NS

_GATHER_1D = lax.GatherDimensionNumbers(
    offset_dims=(), collapsed_slice_dims=(0,), start_index_map=(0,))


def _lane_perm(x, perm):
  return lax.gather(x, perm[:, None], _GATHER_1D, slice_sizes=(1,),
                    mode=lax.GatherScatterMode.PROMISE_IN_BOUNDS)


def _sc_embed_ln(ids, tts, word_table, pos_table, tok_table, gamma, beta,
                 *, seq_len, chunk):
  n_tok = ids.shape[0]
  dim = word_table.shape[1]
  per_w = n_tok // NW
  n_chunks = per_w // chunk
  nvec = dim // LANES
  inv_dim = 1.0 / dim

  mesh = plsc.VectorSubcoreMesh(
      core_axis_name="c", subcore_axis_name="s",
      num_cores=NC, num_subcores=NS)

  @functools.partial(
      pl.kernel,
      out_type=jax.ShapeDtypeStruct((n_tok, dim), jnp.float32),
      mesh=mesh,
      scratch_types=[
          pltpu.VMEM((chunk,), jnp.int32),
          pltpu.VMEM((chunk,), jnp.int32),
          pltpu.VMEM((chunk, dim), jnp.float32),
          pltpu.VMEM((chunk, dim), jnp.float32),
          pltpu.VMEM((chunk, dim), jnp.float32),
          pltpu.VMEM((dim,), jnp.float32),
          pltpu.VMEM((dim,), jnp.float32),
          pltpu.SemaphoreType.DMA,
      ],
  )
  def body(ids_hbm, tts_hbm, word_hbm, pos_hbm, tok_hbm, gamma_hbm, beta_hbm,
           out_hbm, idx_v, ttv, rows_v, pos_v, tokr_v, gamma_v, beta_v, sem):
    wid = lax.axis_index("s") * NC + lax.axis_index("c")
    base = wid * per_w

    pltpu.sync_copy(gamma_hbm, gamma_v)
    pltpu.sync_copy(beta_hbm, beta_v)

    def chunk_body(c, _):
      tb = pl.multiple_of(base + c * chunk, chunk)
      s_base = pl.multiple_of(lax.rem(tb, seq_len), chunk)

      pltpu.sync_copy(pos_hbm.at[pl.ds(s_base, chunk)], pos_v)
      pltpu.sync_copy(ids_hbm.at[pl.ds(tb, chunk)], idx_v)
      pltpu.sync_copy(tts_hbm.at[pl.ds(tb, chunk)], ttv)
      w_cp = pltpu.async_copy(word_hbm.at[idx_v], rows_v, sem)
      t_cp = pltpu.async_copy(tok_hbm.at[ttv], tokr_v, sem)
      w_cp.wait()
      t_cp.wait()

      def token_body(t, _):
        acc_s = jnp.zeros((LANES,), jnp.float32)
        acc_q = jnp.zeros((LANES,), jnp.float32)
        for j in range(nvec):
          sl = pl.ds(j * LANES, LANES)
          x = rows_v[t, sl] + pos_v[t, sl] + tokr_v[t, sl]
          rows_v[t, sl] = x
          acc_s = acc_s + x
          acc_q = acc_q + x * x
        lanes = lax.iota(jnp.int32, LANES)
        for sh in (8, 4, 2, 1):
          perm = lanes ^ sh
          acc_s = acc_s + _lane_perm(acc_s, perm)
          acc_q = acc_q + _lane_perm(acc_q, perm)
        muv = acc_s * inv_dim
        vv = acc_q * inv_dim - muv * muv + 1e-12
        seed = jnp.int32(0x5F3759DF) - (
            lax.bitcast_convert_type(vv, jnp.int32) >> 1)
        y = lax.bitcast_convert_type(seed, jnp.float32)
        for _ in range(3):
          y = y * (1.5 - 0.5 * vv * y * y)
        for j in range(nvec):
          sl = pl.ds(j * LANES, LANES)
          x = rows_v[t, sl]
          rows_v[t, sl] = (x - muv) * y * gamma_v[sl] + beta_v[sl]
        return 0

      lax.fori_loop(0, chunk, token_body, 0)
      pltpu.sync_copy(rows_v, out_hbm.at[pl.ds(tb, chunk)])
      return 0

    lax.fori_loop(0, n_chunks, chunk_body, 0)

  return body(ids, tts, word_table, pos_table, tok_table, gamma, beta)


def kernel(input_ids, token_type_ids, word_table, pos_table, tok_table,
           gamma, beta):
  b, s = input_ids.shape
  dim = word_table.shape[1]
  ids = input_ids.reshape(b * s).astype(jnp.int32)
  tts = token_type_ids.reshape(b * s).astype(jnp.int32)
  out = _sc_embed_ln(ids, tts, word_table.astype(jnp.float32),
                     pos_table.astype(jnp.float32),
                     tok_table.astype(jnp.float32),
                     gamma.astype(jnp.float32), beta.astype(jnp.float32),
                     seq_len=s, chunk=32)
  return out.reshape(b, s, dim)


# speedup vs baseline: 1.2331x; 1.2331x over previous
import functools

import jax
import jax.numpy as jnp
from jax import lax
from jax.experimental import pallas as pl
from jax.experimental.pallas import tpu as pltpu
from jax.experimental.pallas import tpu_sc as plsc

NC = 2
NS = 16
LANES = 16
NW = NC * NS
CH = 16

_GATHER_1D = lax.GatherDimensionNumbers(
    offset_dims=(), collapsed_slice_dims=(0,), start_index_map=(0,))


def _lane_perm(x, perm):
  return lax.gather(x, perm[:, None], _GATHER_1D, slice_sizes=(1,),
                    mode=lax.GatherScatterMode.PROMISE_IN_BOUNDS)


def _sc_embed_ln(ids, tts, word_table, pos_table, tok_table, gamma, beta,
                 *, seq_len):
  n_tok = ids.shape[0]
  dim = word_table.shape[1]
  per_w = n_tok // NW
  n_chunks = per_w // CH
  nvec = dim // LANES
  inv_dim = 1.0 / dim

  mesh = plsc.VectorSubcoreMesh(
      core_axis_name="c", subcore_axis_name="s",
      num_cores=NC, num_subcores=NS)

  @functools.partial(
      pl.kernel,
      out_type=jax.ShapeDtypeStruct((n_tok, dim), jnp.float32),
      mesh=mesh,
      scratch_types=[
          pltpu.VMEM((per_w,), jnp.int32),
          pltpu.VMEM((per_w,), jnp.int32),
          pltpu.VMEM((CH, dim), jnp.float32),
          pltpu.VMEM((CH, dim), jnp.float32),
          pltpu.VMEM((CH, dim), jnp.float32),
          pltpu.VMEM((CH, dim), jnp.float32),
          pltpu.VMEM((CH, dim), jnp.float32),
          pltpu.VMEM((CH, dim), jnp.float32),
          pltpu.VMEM((2, dim), jnp.float32),
          pltpu.VMEM((dim,), jnp.float32),
          pltpu.VMEM((dim,), jnp.float32),
          pltpu.VMEM((dim,), jnp.float32),
          pltpu.VMEM((dim,), jnp.float32),
          pltpu.SemaphoreType.DMA,
          pltpu.SemaphoreType.DMA,
          pltpu.SemaphoreType.DMA,
          pltpu.SemaphoreType.DMA,
      ],
  )
  def body(ids_hbm, tts_hbm, word_hbm, pos_hbm, tok_hbm, gamma_hbm, beta_hbm,
           out_hbm, idxa, tta, r0, r1, p0, p1, o0, o1, tok2_v,
           tokb_v, tokd_v, gamma_v, beta_v, sg0, sg1, so0, so1):
    wid = lax.axis_index("s") * NC + lax.axis_index("c")
    base = wid * per_w

    pltpu.sync_copy(gamma_hbm, gamma_v)
    pltpu.sync_copy(beta_hbm, beta_v)
    pltpu.sync_copy(tok_hbm, tok2_v)
    pltpu.sync_copy(ids_hbm.at[pl.ds(base, per_w)], idxa)
    pltpu.sync_copy(tts_hbm.at[pl.ds(base, per_w)], tta)
    for j in range(nvec):
      sl = pl.ds(j * LANES, LANES)
      t0 = tok2_v[0, sl]
      tokb_v[sl] = t0
      tokd_v[sl] = tok2_v[1, sl] - t0

    slots = ((r0, p0, o0, sg0, so0), (r1, p1, o1, sg1, so1))

    def g_descs(c, rows, pos, sg):
      tb = base + c * CH
      s_b = lax.rem(tb, seq_len)
      d_pos = pltpu.make_async_copy(pos_hbm.at[pl.ds(s_b, CH)], pos, sg)
      d_wrd = pltpu.make_async_copy(word_hbm.at[idxa.at[pl.ds(c * CH, CH)]],
                                    rows, sg)
      return d_pos, d_wrd

    def issue_g(c, rows, pos, sg):
      for d in g_descs(c, rows, pos, sg):
        d.start()

    def wait_g(c, rows, pos, sg):
      for d in g_descs(c, rows, pos, sg):
        d.wait()

    def out_desc(c, outb, so):
      tb = base + c * CH
      return pltpu.make_async_copy(outb, out_hbm.at[pl.ds(tb, CH)], so)

    def compute(c, rows, pos, outb):
      ttv16 = tta[pl.ds(c * CH, CH)]

      def token_body(t, _):
        ttf = _lane_perm(ttv16, jnp.full((LANES,), t, jnp.int32)).astype(
            jnp.float32)
        acc_s = jnp.zeros((LANES,), jnp.float32)
        acc_q = jnp.zeros((LANES,), jnp.float32)
        for j in range(nvec):
          sl = pl.ds(j * LANES, LANES)
          x = rows[t, sl] + pos[t, sl] + (tokb_v[sl] + ttf * tokd_v[sl])
          outb[t, sl] = x
          acc_s = acc_s + x
          acc_q = acc_q + x * x
        lanes = lax.iota(jnp.int32, LANES)
        for sh in (8, 4, 2, 1):
          perm = lanes ^ sh
          acc_s = acc_s + _lane_perm(acc_s, perm)
          acc_q = acc_q + _lane_perm(acc_q, perm)
        muv = acc_s * inv_dim
        vv = acc_q * inv_dim - muv * muv + 1e-12
        seed = jnp.int32(0x5F3759DF) - (
            lax.bitcast_convert_type(vv, jnp.int32) >> 1)
        y = lax.bitcast_convert_type(seed, jnp.float32)
        for _ in range(3):
          y = y * (1.5 - 0.5 * vv * y * y)
        for j in range(nvec):
          sl = pl.ds(j * LANES, LANES)
          x = outb[t, sl]
          outb[t, sl] = (x - muv) * y * gamma_v[sl] + beta_v[sl]
        return 0

      lax.fori_loop(0, CH, token_body, 0)

    issue_g(0, r0, p0, sg0)
    issue_g(1, r1, p1, sg1)

    def pair_body(k, _):
      for b in (0, 1):
        rows, pos, outb, sg, so = slots[b]
        c = 2 * k + b
        wait_g(c, rows, pos, sg)

        @pl.when(c >= 2)
        def _():
          out_desc(c, outb, so).wait()

        compute(c, rows, pos, outb)
        out_desc(c, outb, so).start()

        @pl.when(c + 2 < n_chunks)
        def _():
          issue_g(c + 2, rows, pos, sg)
      return 0

    lax.fori_loop(0, n_chunks // 2, pair_body, 0)
    out_desc(n_chunks - 2, o0, so0).wait()
    out_desc(n_chunks - 1, o1, so1).wait()

  return body(ids, tts, word_table, pos_table, tok_table, gamma, beta)


def kernel(input_ids, token_type_ids, word_table, pos_table, tok_table,
           gamma, beta):
  b, s = input_ids.shape
  dim = word_table.shape[1]
  ids = input_ids.reshape(b * s).astype(jnp.int32)
  tts = token_type_ids.reshape(b * s).astype(jnp.int32)
  out = _sc_embed_ln(ids, tts, word_table.astype(jnp.float32),
                     pos_table.astype(jnp.float32),
                     tok_table.astype(jnp.float32),
                     gamma.astype(jnp.float32), beta.astype(jnp.float32),
                     seq_len=s)
  return out.reshape(b, s, dim)


# speedup vs baseline: 3.0563x; 2.4786x over previous
import functools

import jax
import jax.numpy as jnp
from jax import lax
from jax.experimental import pallas as pl
from jax.experimental.pallas import tpu as pltpu
from jax.experimental.pallas import tpu_sc as plsc

NC = 2
NS = 16
LANES = 16
NW = NC * NS
CH = 16

_GATHER_1D = lax.GatherDimensionNumbers(
    offset_dims=(), collapsed_slice_dims=(0,), start_index_map=(0,))


def _lane_perm(x, perm):
  return lax.gather(x, perm[:, None], _GATHER_1D, slice_sizes=(1,),
                    mode=lax.GatherScatterMode.PROMISE_IN_BOUNDS)


def _sc_embed_ln(ids, tts, word_table, pos_table, tok_table, gamma, beta,
                 *, seq_len):
  n_tok = ids.shape[0]
  dim = word_table.shape[1]
  per_w = n_tok // NW
  n_chunks = per_w // CH
  nvec = dim // LANES
  inv_dim = 1.0 / dim

  mesh = plsc.VectorSubcoreMesh(
      core_axis_name="c", subcore_axis_name="s",
      num_cores=NC, num_subcores=NS)

  @functools.partial(
      pl.kernel,
      out_type=jax.ShapeDtypeStruct((n_tok, dim), jnp.float32),
      mesh=mesh,
      scratch_types=[
          pltpu.VMEM((per_w,), jnp.int32),
          pltpu.VMEM((per_w,), jnp.int32),
          pltpu.VMEM((CH, dim), jnp.float32),
          pltpu.VMEM((CH, dim), jnp.float32),
          pltpu.VMEM((CH, dim), jnp.float32),
          pltpu.VMEM((CH, dim), jnp.float32),
          pltpu.VMEM((CH, dim), jnp.float32),
          pltpu.VMEM((CH, dim), jnp.float32),
          pltpu.VMEM((2, dim), jnp.float32),
          pltpu.VMEM((dim,), jnp.float32),
          pltpu.VMEM((dim,), jnp.float32),
          pltpu.VMEM((dim,), jnp.float32),
          pltpu.VMEM((dim,), jnp.float32),
          pltpu.SemaphoreType.DMA,
          pltpu.SemaphoreType.DMA,
          pltpu.SemaphoreType.DMA,
          pltpu.SemaphoreType.DMA,
      ],
  )
  def body(ids_hbm, tts_hbm, word_hbm, pos_hbm, tok_hbm, gamma_hbm, beta_hbm,
           out_hbm, idxa, tta, r0, r1, p0, p1, o0, o1, tok2_v,
           tokb_v, tokd_v, gamma_v, beta_v, sg0, sg1, so0, so1):
    wid = lax.axis_index("s") * NC + lax.axis_index("c")
    base = wid * per_w

    pltpu.sync_copy(gamma_hbm, gamma_v)
    pltpu.sync_copy(beta_hbm, beta_v)
    pltpu.sync_copy(tok_hbm, tok2_v)
    pltpu.sync_copy(ids_hbm.at[pl.ds(base, per_w)], idxa)
    pltpu.sync_copy(tts_hbm.at[pl.ds(base, per_w)], tta)
    for j in range(nvec):
      sl = pl.ds(j * LANES, LANES)
      t0 = tok2_v[0, sl]
      tokb_v[sl] = t0
      tokd_v[sl] = tok2_v[1, sl] - t0

    slots = ((r0, p0, o0, sg0, so0), (r1, p1, o1, sg1, so1))

    def g_descs(c, rows, pos, sg):
      tb = base + c * CH
      s_b = lax.rem(tb, seq_len)
      d_pos = pltpu.make_async_copy(pos_hbm.at[pl.ds(s_b, CH)], pos, sg)
      d_wrd = pltpu.make_async_copy(word_hbm.at[idxa.at[pl.ds(c * CH, CH)]],
                                    rows, sg)
      return d_pos, d_wrd

    def issue_g(c, rows, pos, sg):
      for d in g_descs(c, rows, pos, sg):
        d.start()

    def wait_g(c, rows, pos, sg):
      for d in g_descs(c, rows, pos, sg):
        d.wait()

    def out_desc(c, outb, so):
      tb = base + c * CH
      return pltpu.make_async_copy(outb, out_hbm.at[pl.ds(tb, CH)], so)

    GRP = 4
    n_grp = nvec // GRP

    def compute(c, rows, pos, outb):
      ttv16 = tta[pl.ds(c * CH, CH)]

      def token_body(t, _):
        ttf = _lane_perm(ttv16, jnp.full((LANES,), t, jnp.int32)).astype(
            jnp.float32)

        def load1(g):
          out = []
          for u in range(GRP):
            sl = pl.ds((g * GRP + u) * LANES, LANES)
            out.append((rows[t, sl], pos[t, sl], tokb_v[sl], tokd_v[sl], sl))
          return out

        accs = [jnp.zeros((LANES,), jnp.float32) for _ in range(GRP)]
        accq = [jnp.zeros((LANES,), jnp.float32) for _ in range(GRP)]

        def consume1(vals):
          for u, (w, p, tb, td, sl) in enumerate(vals):
            x = (w + p) + (tb + ttf * td)
            outb[t, sl] = x
            accs[u] = accs[u] + x
            accq[u] = accq[u] + x * x

        prev = load1(0)
        for g in range(1, n_grp):
          cur = load1(g)
          consume1(prev)
          prev = cur
        consume1(prev)

        acc_s = (accs[0] + accs[1]) + (accs[2] + accs[3])
        acc_q = (accq[0] + accq[1]) + (accq[2] + accq[3])
        lanes = lax.iota(jnp.int32, LANES)
        for sh in (8, 4, 2, 1):
          perm = lanes ^ sh
          acc_s = acc_s + _lane_perm(acc_s, perm)
          acc_q = acc_q + _lane_perm(acc_q, perm)
        muv = acc_s * inv_dim
        vv = acc_q * inv_dim - muv * muv + 1e-12
        seed = jnp.int32(0x5F3759DF) - (
            lax.bitcast_convert_type(vv, jnp.int32) >> 1)
        y = lax.bitcast_convert_type(seed, jnp.float32)
        for _ in range(2):
          y = y * (1.5 - 0.5 * vv * y * y)

        def load2(g):
          out = []
          for u in range(GRP):
            sl = pl.ds((g * GRP + u) * LANES, LANES)
            out.append((outb[t, sl], gamma_v[sl], beta_v[sl], sl))
          return out

        def consume2(vals):
          for x, gmm, bta, sl in vals:
            outb[t, sl] = ((x - muv) * y) * gmm + bta

        prev = load2(0)
        for g in range(1, n_grp):
          cur = load2(g)
          consume2(prev)
          prev = cur
        consume2(prev)
        return 0

      lax.fori_loop(0, CH, token_body, 0)

    issue_g(0, r0, p0, sg0)
    issue_g(1, r1, p1, sg1)

    def pair_body(k, _):
      for b in (0, 1):
        rows, pos, outb, sg, so = slots[b]
        c = 2 * k + b
        wait_g(c, rows, pos, sg)

        @pl.when(c >= 2)
        def _():
          out_desc(c, outb, so).wait()

        compute(c, rows, pos, outb)
        out_desc(c, outb, so).start()

        @pl.when(c + 2 < n_chunks)
        def _():
          issue_g(c + 2, rows, pos, sg)
      return 0

    lax.fori_loop(0, n_chunks // 2, pair_body, 0)
    out_desc(n_chunks - 2, o0, so0).wait()
    out_desc(n_chunks - 1, o1, so1).wait()

  return body(ids, tts, word_table, pos_table, tok_table, gamma, beta)


def kernel(input_ids, token_type_ids, word_table, pos_table, tok_table,
           gamma, beta):
  b, s = input_ids.shape
  dim = word_table.shape[1]
  ids = input_ids.reshape(b * s).astype(jnp.int32)
  tts = token_type_ids.reshape(b * s).astype(jnp.int32)
  out = _sc_embed_ln(ids, tts, word_table.astype(jnp.float32),
                     pos_table.astype(jnp.float32),
                     tok_table.astype(jnp.float32),
                     gamma.astype(jnp.float32), beta.astype(jnp.float32),
                     seq_len=s)
  return out.reshape(b, s, dim)
